# baseline (speedup 1.0000x reference)
import functools

import jax
import jax.numpy as jnp
from jax import lax
from jax.experimental import pallas as pl
from jax.experimental.pallas import tpu as pltpu
from jax.experimental.pallas import tpu_sc as plsc

B, N, M, C = 2, 1024, 1024, 128
DIM, K, PHD, HID = 256, 16, 64, 1024
NK = N * K
EPS = 1e-5
F32 = jnp.float32
I32 = jnp.int32

NC, NS = 2, 16
NW = NC * NS
RPW = (B * N) // NW

_SC_MESH = plsc.VectorSubcoreMesh(core_axis_name="c", subcore_axis_name="s")


def _prep_body(pos_ref, qimT_ref, kpcT_ref, kpc_ref, valueT_ref,
               WkT_ref, bk_ref, WvT_ref, bv_ref, vw1T_ref, vb1_ref,
               keyT_ref, valT_ref, H1T_ref, cos_ref, negd2_ref):
    keyT = jnp.dot(kpcT_ref[...], WkT_ref[...],
                   preferred_element_type=F32) + bk_ref[...]
    valT = jnp.dot(valueT_ref[...], WvT_ref[...],
                   preferred_element_type=F32) + bv_ref[...]
    keyT_ref[...] = keyT
    valT_ref[...] = valT
    H1T_ref[...] = jnp.dot(keyT, vw1T_ref[...],
                           preferred_element_type=F32) + vb1_ref[...]
    qimT = qimT_ref[...]
    kpc = kpc_ref[...]
    num = jnp.dot(qimT, kpc, preferred_element_type=F32)
    qn = jnp.sqrt(jnp.sum(qimT * qimT, axis=1, keepdims=True))
    kn = jnp.sqrt(jnp.sum(kpc * kpc, axis=0, keepdims=True))
    cos_ref[...] = num / (qn * kn)
    acc = jnp.zeros((N, N), F32)
    for c in range(3):
        pc = pos_ref[c, :]
        d = pc[:, None] - pc[None, :]
        acc = acc + d * d
    negd2_ref[...] = -acc


def _prep(pos, qimT, kpcT, kpc, valueT, WkT, bk2, WvT, bv2, vw1T, vb12):
    full = lambda s: pl.BlockSpec((1,) + s, lambda b: (b, 0, 0))
    w = lambda s: pl.BlockSpec(s, lambda b: (0,) * len(s))
    specs_in = [full((3, N)), full((M, C)), full((N, C)), full((C, N)),
                full((N, C)), w((C, DIM)), w((1, DIM)), w((C, DIM)),
                w((1, DIM)), w((DIM, HID)), w((1, HID))]
    out_shapes = [jax.ShapeDtypeStruct((B, N, DIM), F32),
                  jax.ShapeDtypeStruct((B, N, DIM), F32),
                  jax.ShapeDtypeStruct((B, N, HID), F32),
                  jax.ShapeDtypeStruct((B, M, N), F32),
                  jax.ShapeDtypeStruct((B, N, N), F32)]
    specs_out = [full((N, DIM)), full((N, DIM)), full((N, HID)),
                 full((M, N)), full((N, N))]

    def body(pos_r, qimT_r, kpcT_r, kpc_r, valueT_r, WkT_r, bk_r, WvT_r,
             bv_r, vw1T_r, vb1_r, keyT_r, valT_r, H1T_r, cos_r, negd2_r):
        _prep_body(pos_r.at[0], qimT_r.at[0], kpcT_r.at[0], kpc_r.at[0],
                   valueT_r.at[0], WkT_r, bk_r, WvT_r, bv_r, vw1T_r, vb1_r,
                   keyT_r.at[0], valT_r.at[0], H1T_r.at[0], cos_r.at[0],
                   negd2_r.at[0])

    return pl.pallas_call(
        body, grid=(B,), in_specs=specs_in, out_specs=specs_out,
        out_shape=out_shapes)(pos, qimT, kpcT, kpc, valueT, WkT, bk2, WvT,
                              bv2, vw1T, vb12)


_TOPK_TILE = 256


def _topk_body(x_ref, idx_ref):
    x = x_ref[...]
    it = lax.broadcasted_iota(I32, (_TOPK_TILE, N), 1)
    cols = []
    for _ in range(K):
        m = jnp.max(x, axis=1, keepdims=True)
        cand = jnp.where(x == m, it, N)
        am = jnp.min(cand, axis=1, keepdims=True)
        cols.append(am)
        x = jnp.where(it == am, -jnp.inf, x)
    idx_ref[...] = jnp.concatenate(cols, axis=1)


def _topk(scores):
    rows = scores.shape[0]
    grid = (rows // _TOPK_TILE,)
    return pl.pallas_call(
        _topk_body, grid=grid,
        in_specs=[pl.BlockSpec((_TOPK_TILE, N), lambda i: (i, 0))],
        out_specs=pl.BlockSpec((_TOPK_TILE, K), lambda i: (i, 0)),
        out_shape=jax.ShapeDtypeStruct((rows, K), I32))(scores)


_GCH = 8
_GROWS = 128


def _sc_gather_body(keyT_hbm, posT_hbm, ig_hbm, keyg_hbm, posg_hbm,
                    idx_v, kb0, kb1, pb0, pb1, semg, sems):
    wid = lax.axis_index("s") * NC + lax.axis_index("c")
    base = wid * (_GCH * _GROWS)
    pltpu.sync_copy(ig_hbm.at[pl.ds(base, _GCH * _GROWS)], idx_v)
    kb = [kb0, kb1]
    pb = [pb0, pb1]
    sd = [None] * _GCH
    for c in range(_GCH):
        if c >= 2:
            sd[c - 2][0].wait()
            sd[c - 2][1].wait()
        isl = idx_v.at[pl.ds(c * _GROWS, _GROWS)]
        g1 = pltpu.async_copy(keyT_hbm.at[isl], kb[c % 2], semg)
        g2 = pltpu.async_copy(posT_hbm.at[isl], pb[c % 2], semg)
        g1.wait()
        g2.wait()
        s1 = pltpu.async_copy(kb[c % 2],
                              keyg_hbm.at[pl.ds(base + c * _GROWS, _GROWS)],
                              sems)
        s2 = pltpu.async_copy(pb[c % 2],
                              posg_hbm.at[pl.ds(base + c * _GROWS, _GROWS)],
                              sems)
        sd[c] = (s1, s2)
    for c in (_GCH - 2, _GCH - 1):
        sd[c][0].wait()
        sd[c][1].wait()


@functools.partial(
    pl.kernel, mesh=_SC_MESH,
    out_type=[jax.ShapeDtypeStruct((B * N * K, DIM), F32),
              jax.ShapeDtypeStruct((B * N * K, 128), F32)],
    scratch_types=[pltpu.VMEM((_GCH * _GROWS,), I32),
                   pltpu.VMEM((_GROWS, DIM), F32),
                   pltpu.VMEM((_GROWS, DIM), F32),
                   pltpu.VMEM((_GROWS, 128), F32),
                   pltpu.VMEM((_GROWS, 128), F32),
                   pltpu.SemaphoreType.DMA,
                   pltpu.SemaphoreType.DMA])
def _sc_gather(keyT_hbm, posT_hbm, ig_hbm, keyg_hbm, posg_hbm,
               idx_v, kb0, kb1, pb0, pb1, semg, sems):
    _sc_gather_body(keyT_hbm, posT_hbm, ig_hbm, keyg_hbm, posg_hbm,
                    idx_v, kb0, kb1, pb0, pb1, semg, sems)


_HIST_TILE = 256


def _hist_body(idx_r, cnt_r):
    it = lax.broadcasted_iota(I32, (_HIST_TILE, B * N), 1)
    acc = jnp.zeros((1, B * N), F32)
    idx = idx_r[...]
    for k in range(K):
        eq = (idx[:, k:k + 1] == it).astype(F32)
        acc = acc + jnp.sum(eq, axis=0, keepdims=True)
    blk = jnp.concatenate([acc, jnp.zeros((7, B * N), F32)], axis=0)

    @pl.when(pl.program_id(0) == 0)
    def _():
        cnt_r[...] = blk

    @pl.when(pl.program_id(0) != 0)
    def _():
        cnt_r[...] = cnt_r[...] + blk


def _hist(iv_off):
    grid = ((B * M) // _HIST_TILE,)
    return pl.pallas_call(
        _hist_body, grid=grid,
        in_specs=[pl.BlockSpec((_HIST_TILE, K), lambda i: (i, 0))],
        out_specs=pl.BlockSpec((8, B * N), lambda i: (0, 0)),
        out_shape=jax.ShapeDtypeStruct((8, B * N), F32))(iv_off)


_ACH = 16
_AM = 4


def _attn_rows(lvb, out_v, orow, jm):
    r0 = jm * K
    for g in range(DIM // 16):
        sl = pl.ds(g * 16, 16)
        sv = pl.ds(DIM + g * 16, 16)
        ls = [lvb[r0 + k, sl] for k in range(K)]
        m = ls[0]
        for k in range(1, K):
            m = jnp.maximum(m, ls[k])
        es = [jnp.exp(ls[k] - m) for k in range(K)]
        ssum = es[0]
        for k in range(1, K):
            ssum = ssum + es[k]
        acc = es[0] * lvb[r0, sv]
        for k in range(1, K):
            acc = acc + es[k] * lvb[r0 + k, sv]
        out_v[orow, sl] = acc / ssum


def _sc_attn_body(LVT_hbm, iv_hbm, agg_hbm,
                  idx_v, lvb0, lvb1, out_v, semg):
    wid = lax.axis_index("s") * NC + lax.axis_index("c")
    base = wid * RPW
    rows = _AM * K
    pltpu.sync_copy(iv_hbm.at[pl.ds(base * K, RPW * K)], idx_v)
    pltpu.async_copy(LVT_hbm.at[idx_v.at[pl.ds(0, rows)]], lvb0, semg)

    dummy = LVT_hbm.at[pl.ds(0, rows)]

    def pair(j, carry):
        c0 = 2 * j
        c1 = 2 * j + 1
        pltpu.make_async_copy(dummy, lvb0, semg).wait()
        i1 = idx_v.at[pl.ds(c1 * rows, rows)]
        pltpu.async_copy(LVT_hbm.at[i1], lvb1, semg)

        def m0(jm, cc):
            _attn_rows(lvb0, out_v, c0 * _AM + jm, jm)
            return cc
        lax.fori_loop(0, _AM, m0, 0)

        pltpu.make_async_copy(dummy, lvb1, semg).wait()
        cn = lax.rem(c0 + 2, _ACH)
        i2 = idx_v.at[pl.ds(cn * rows, rows)]
        pltpu.async_copy(LVT_hbm.at[i2], lvb0, semg)

        def m1(jm, cc):
            _attn_rows(lvb1, out_v, c1 * _AM + jm, jm)
            return cc
        lax.fori_loop(0, _AM, m1, 0)
        return carry
    lax.fori_loop(0, _ACH // 2, pair, 0)
    pltpu.make_async_copy(dummy, lvb0, semg).wait()
    pltpu.sync_copy(out_v, agg_hbm.at[pl.ds(base, RPW)])


@functools.partial(
    pl.kernel, mesh=_SC_MESH,
    out_type=jax.ShapeDtypeStruct((B * M, DIM), F32),
    scratch_types=[pltpu.VMEM((RPW * K,), I32),
                   pltpu.VMEM((_AM * K, 2 * DIM), F32),
                   pltpu.VMEM((_AM * K, 2 * DIM), F32),
                   pltpu.VMEM((RPW, DIM), F32),
                   pltpu.SemaphoreType.DMA])
def _sc_attn(LVT_hbm, iv_hbm, agg_hbm, idx_v, lvb0, lvb1, out_v, semg):
    _sc_attn_body(LVT_hbm, iv_hbm, agg_hbm, idx_v, lvb0, lvb1, out_v,
                  semg)


def _bstats_body(cnt_r, H1T_r, s_r):
    cv = cnt_r[0:1, :]
    h = H1T_r.at[0][...]
    s1 = jnp.dot(cv, h, preferred_element_type=F32)
    s2 = jnp.dot(cv, h * h, preferred_element_type=F32)
    blk = jnp.concatenate([s1, s2, jnp.zeros((6, HID), F32)], axis=0)

    @pl.when(pl.program_id(0) == 0)
    def _():
        s_r[...] = blk

    @pl.when(pl.program_id(0) != 0)
    def _():
        s_r[...] = s_r[...] + blk


def _bstats(cnt_part, H1T):
    return pl.pallas_call(
        _bstats_body, grid=(B,),
        in_specs=[pl.BlockSpec((8, N), lambda b: (0, b)),
                  pl.BlockSpec((1, N, HID), lambda b: (b, 0, 0))],
        out_specs=pl.BlockSpec((8, HID), lambda b: (0, 0)),
        out_shape=jax.ShapeDtypeStruct((8, HID), F32))(cnt_part, H1T)


def _lview_body(H1T_r, s_r, vg_r, vbe_r, vw2T_r, vb2_r, LT_r):
    stot = float(B * M * K)
    mu = s_r[0:1, :] / stot
    var = s_r[1:2, :] / stot - mu * mu
    a = vg_r[...] / jnp.sqrt(var + EPS)
    c = vbe_r[...] - mu * a
    H = jnp.maximum(H1T_r.at[0][...] * a + c, 0.0)
    LT_r.at[0][...] = jnp.dot(H, vw2T_r[...],
                              preferred_element_type=F32) + vb2_r[...]


def _lview(H1T, s, vg2, vbe2, vw2T, vb22):
    return pl.pallas_call(
        _lview_body, grid=(B,),
        in_specs=[pl.BlockSpec((1, N, HID), lambda b: (b, 0, 0)),
                  pl.BlockSpec((8, HID), lambda b: (0, 0)),
                  pl.BlockSpec((1, HID), lambda b: (0, 0)),
                  pl.BlockSpec((1, HID), lambda b: (0, 0)),
                  pl.BlockSpec((HID, DIM), lambda b: (0, 0)),
                  pl.BlockSpec((1, DIM), lambda b: (0, 0))],
        out_specs=pl.BlockSpec((1, N, DIM), lambda b: (b, 0, 0)),
        out_shape=jax.ShapeDtypeStruct((B, N, DIM), F32))(
            H1T, s, vg2, vbe2, vw2T, vb22)


def _geo1_body(posg_r, posTp_r, W2_r, pb1_r, pw2Tp_r, gw1T_r, pb2_r,
               gb1_r, p1_r, sp_r, GPT_r, cb_r):
    own = posTp_r.at[0][...]
    pg_ = posg_r.at[0][...]
    posd = (own[:, None, :] - pg_.reshape(N, K, 128)).reshape(NK, 128)
    p1 = jnp.dot(posd, W2_r[...],
                 preferred_element_type=F32) + pb1_r[...]
    p1_r.at[0][...] = p1
    s1 = jnp.sum(p1, axis=0, keepdims=True)
    s2 = jnp.sum(p1 * p1, axis=0, keepdims=True)
    blk = jnp.concatenate([s1, s2, jnp.zeros((6, 128), F32)], axis=0)

    @pl.when(pl.program_id(0) == 0)
    def _():
        sp_r[...] = blk
        GPT_r[...] = jnp.dot(pw2Tp_r[...], gw1T_r[...],
                             preferred_element_type=F32)
        cb_r[...] = jnp.dot(pb2_r[...], gw1T_r[...],
                            preferred_element_type=F32) + gb1_r[...]

    @pl.when(pl.program_id(0) != 0)
    def _():
        sp_r[...] = sp_r[...] + blk


def _geo1(posg3, posTp, W2, pb1p, pw2Tp, gw1T, pb22, gb12):
    return pl.pallas_call(
        _geo1_body, grid=(B,),
        in_specs=[pl.BlockSpec((1, NK, 128), lambda b: (b, 0, 0)),
                  pl.BlockSpec((1, N, 128), lambda b: (b, 0, 0)),
                  pl.BlockSpec((128, 128), lambda b: (0, 0)),
                  pl.BlockSpec((1, 128), lambda b: (0, 0)),
                  pl.BlockSpec((128, DIM), lambda b: (0, 0)),
                  pl.BlockSpec((DIM, HID), lambda b: (0, 0)),
                  pl.BlockSpec((1, DIM), lambda b: (0, 0)),
                  pl.BlockSpec((1, HID), lambda b: (0, 0))],
        out_specs=[pl.BlockSpec((1, NK, 128), lambda b: (b, 0, 0)),
                   pl.BlockSpec((8, 128), lambda b: (0, 0)),
                   pl.BlockSpec((128, HID), lambda b: (0, 0)),
                   pl.BlockSpec((1, HID), lambda b: (0, 0))],
        out_shape=[jax.ShapeDtypeStruct((B, NK, 128), F32),
                   jax.ShapeDtypeStruct((8, 128), F32),
                   jax.ShapeDtypeStruct((128, HID), F32),
                   jax.ShapeDtypeStruct((1, HID), F32)])(
            posg3, posTp, W2, pb1p, pw2Tp, gw1T, pb22, gb12)


_G2T = 1024


def _geo2_body(keyg_r, ownk_r, p1_r, sp_r, gw1T_r, GPT_r, cb_r, pg_r,
               pbe_r, sg_r):
    stot = float(B * NK)
    mup = sp_r[0:1, :] / stot
    varp = sp_r[1:2, :] / stot - mup * mup
    ap = pg_r[...] / jnp.sqrt(varp + EPS)
    cp = pbe_r[...] - mup * ap
    pe = jnp.maximum(p1_r.at[0][...] * ap + cp, 0.0)
    own = ownk_r.at[0][...]
    diff = (own[:, None, :]
            - keyg_r.at[0][...].reshape(_G2T // K, K, DIM)).reshape(
                _G2T, DIM)
    pre1 = (jnp.dot(diff, gw1T_r[...],
                    preferred_element_type=F32)
            + jnp.dot(pe, GPT_r[...], preferred_element_type=F32)
            + cb_r[...])
    s1 = jnp.sum(pre1, axis=0, keepdims=True)
    s2 = jnp.sum(pre1 * pre1, axis=0, keepdims=True)
    blk = jnp.concatenate([s1, s2, jnp.zeros((6, HID), F32)], axis=0)
    first = jnp.logical_and(pl.program_id(0) == 0, pl.program_id(1) == 0)

    @pl.when(first)
    def _():
        sg_r[...] = blk

    @pl.when(jnp.logical_not(first))
    def _():
        sg_r[...] = sg_r[...] + blk


def _geo2(keyg3, keyT, p1, sp, gw1T, GPT, cb, pgp, pbep):
    nt = NK // _G2T
    return pl.pallas_call(
        _geo2_body, grid=(B, nt),
        in_specs=[pl.BlockSpec((1, _G2T, DIM), lambda b, t: (b, t, 0)),
                  pl.BlockSpec((1, _G2T // K, DIM), lambda b, t: (b, t, 0)),
                  pl.BlockSpec((1, _G2T, 128), lambda b, t: (b, t, 0)),
                  pl.BlockSpec((8, 128), lambda b, t: (0, 0)),
                  pl.BlockSpec((DIM, HID), lambda b, t: (0, 0)),
                  pl.BlockSpec((128, HID), lambda b, t: (0, 0)),
                  pl.BlockSpec((1, HID), lambda b, t: (0, 0)),
                  pl.BlockSpec((1, 128), lambda b, t: (0, 0)),
                  pl.BlockSpec((1, 128), lambda b, t: (0, 0))],
        out_specs=pl.BlockSpec((8, HID), lambda b, t: (0, 0)),
        out_shape=jax.ShapeDtypeStruct((8, HID), F32))(
            keyg3, keyT, p1, sp, gw1T, GPT, cb, pgp, pbep)


def _geo3_body(keyg_r, ownk_r, p1_r, sg_r, sp_r, valT_r, gw1T_r, GPT_r,
               cb_r, gw2T_r, gb2_r, pw2Tp_r, pb2_r, gg_r, gbe_r, pg_r,
               pbe_r, fg_r):
    stot = float(B * NK)
    mup = sp_r[0:1, :] / stot
    varp = sp_r[1:2, :] / stot - mup * mup
    ap = pg_r[...] / jnp.sqrt(varp + EPS)
    cp = pbe_r[...] - mup * ap
    pe = jnp.maximum(p1_r.at[0][...] * ap + cp, 0.0)
    own = ownk_r.at[0][...]
    diff = (own[:, None, :]
            - keyg_r.at[0][...].reshape(_G2T // K, K, DIM)).reshape(
                _G2T, DIM)
    pre1 = (jnp.dot(diff, gw1T_r[...], preferred_element_type=F32)
            + jnp.dot(pe, GPT_r[...], preferred_element_type=F32)
            + cb_r[...])
    mug = sg_r[0:1, :] / stot
    varg = sg_r[1:2, :] / stot - mug * mug
    ag = gg_r[...] / jnp.sqrt(varg + EPS)
    cg = gbe_r[...] - mug * ag
    hg = jnp.maximum(pre1 * ag + cg, 0.0)
    logits = jnp.dot(hg, gw2T_r[...],
                     preferred_element_type=F32) + gb2_r[...]
    lg = logits.reshape(_G2T // K, K, DIM)
    mx = jnp.max(lg, axis=1, keepdims=True)
    e = jnp.exp(lg - mx)
    attn = e / jnp.sum(e, axis=1, keepdims=True)
    pos_emb = (jnp.dot(pe, pw2Tp_r[...], preferred_element_type=F32)
               + pb2_r[...]).reshape(_G2T // K, K, DIM)
    v = valT_r.at[0][...]
    fg_r.at[0][...] = (jnp.sum(attn, axis=1) * v
                       + jnp.sum(attn * pos_emb, axis=1))


def _geo3(keyg3, keyT, p1, sg, sp, valT, gw1T, GPT, cb, gw2T, gb22,
          pw2Tp, pb22, gg2, gbe2, pgp, pbep):
    nt = NK // _G2T
    npts = _G2T // K
    return pl.pallas_call(
        _geo3_body, grid=(B, nt),
        in_specs=[pl.BlockSpec((1, _G2T, DIM), lambda b, t: (b, t, 0)),
                  pl.BlockSpec((1, npts, DIM), lambda b, t: (b, t, 0)),
                  pl.BlockSpec((1, _G2T, 128), lambda b, t: (b, t, 0)),
                  pl.BlockSpec((8, HID), lambda b, t: (0, 0)),
                  pl.BlockSpec((8, 128), lambda b, t: (0, 0)),
                  pl.BlockSpec((1, npts, DIM), lambda b, t: (b, t, 0)),
                  pl.BlockSpec((DIM, HID), lambda b, t: (0, 0)),
                  pl.BlockSpec((128, HID), lambda b, t: (0, 0)),
                  pl.BlockSpec((1, HID), lambda b, t: (0, 0)),
                  pl.BlockSpec((HID, DIM), lambda b, t: (0, 0)),
                  pl.BlockSpec((1, DIM), lambda b, t: (0, 0)),
                  pl.BlockSpec((128, DIM), lambda b, t: (0, 0)),
                  pl.BlockSpec((1, DIM), lambda b, t: (0, 0)),
                  pl.BlockSpec((1, HID), lambda b, t: (0, 0)),
                  pl.BlockSpec((1, HID), lambda b, t: (0, 0)),
                  pl.BlockSpec((1, 128), lambda b, t: (0, 0)),
                  pl.BlockSpec((1, 128), lambda b, t: (0, 0))],
        out_specs=pl.BlockSpec((1, npts, DIM), lambda b, t: (b, t, 0)),
        out_shape=jax.ShapeDtypeStruct((B, N, DIM), F32))(
            keyg3, keyT, p1, sg, sp, valT, gw1T, GPT, cb, gw2T, gb22,
            pw2Tp, pb22, gg2, gbe2, pgp, pbep)


def _final_body(cos_r, aggT_r, fgT_r, value_r, Wview_r, bview_r, Wgeo_r,
                bgeo_r, out_r):
    cos = cos_r.at[0][...]
    mx = jnp.max(cos, axis=0, keepdims=True)
    e = jnp.exp(cos - mx)
    s = jnp.sum(e, axis=0, keepdims=True)
    e = e / s
    fvT = lax.dot_general(e, aggT_r.at[0][...],
                          (((0,), (0,)), ((), ())),
                          preferred_element_type=F32)
    pv = lax.dot_general(Wview_r[...], fvT, (((1,), (1,)), ((), ())),
                         preferred_element_type=F32)
    pg_ = lax.dot_general(Wgeo_r[...], fgT_r.at[0][...],
                          (((1,), (1,)), ((), ())),
                          preferred_element_type=F32)
    out_r.at[0][...] = (value_r.at[0][...] + pv + bview_r[...]
                        + pg_ + bgeo_r[...])


def _final(cos, aggT, fgT, value, Wview, bviewC, Wgeo, bgeoC):
    return pl.pallas_call(
        _final_body, grid=(B,),
        in_specs=[pl.BlockSpec((1, M, N), lambda b: (b, 0, 0)),
                  pl.BlockSpec((1, M, DIM), lambda b: (b, 0, 0)),
                  pl.BlockSpec((1, N, DIM), lambda b: (b, 0, 0)),
                  pl.BlockSpec((1, C, N), lambda b: (b, 0, 0)),
                  pl.BlockSpec((C, DIM), lambda b: (0, 0)),
                  pl.BlockSpec((C, 1), lambda b: (0, 0)),
                  pl.BlockSpec((C, DIM), lambda b: (0, 0)),
                  pl.BlockSpec((C, 1), lambda b: (0, 0))],
        out_specs=pl.BlockSpec((1, C, N), lambda b: (b, 0, 0)),
        out_shape=jax.ShapeDtypeStruct((B, C, N), F32))(
            cos, aggT, fgT, value, Wview, bviewC, Wgeo, bgeoC)


def kernel(pos, query_im, key_pc, value, Wk, bk, Wv, bv,
           pw1, pb1, pg, pbe, pw2, pb2,
           vw1, vb1, vg, vbe, vw2, vb2,
           gw1, gb1, gg, gbe, gw2, gb2,
           Wview, bview, Wgeo, bgeo):
    qimT = jnp.transpose(query_im, (0, 2, 1))
    kpcT = jnp.transpose(key_pc, (0, 2, 1))
    valueT = jnp.transpose(value, (0, 2, 1))
    posT = jnp.transpose(pos, (0, 2, 1))
    posTp = jnp.pad(posT, ((0, 0), (0, 0), (0, 125)))

    row = lambda x: x.reshape(1, -1)
    keyT, valT, H1T, cos, negd2 = _prep(
        pos, qimT, kpcT, key_pc, valueT, Wk.T, row(bk), Wv.T, row(bv),
        vw1.T, row(vb1))

    scores = jnp.concatenate(
        [cos.reshape(B * M, N), negd2.reshape(B * N, N)], axis=0)
    idx_all = _topk(scores)
    off = ((jnp.arange(B * N, dtype=I32) // N) * N)[:, None]
    iv_off = idx_all[:B * M] + off
    ig_off = idx_all[B * M:] + off

    keyT2 = keyT.reshape(B * N, DIM)
    valT2 = valT.reshape(B * N, DIM)
    posT2 = posTp.reshape(B * N, 128)
    keygK, posgK = _sc_gather(keyT2, posT2, ig_off.reshape(-1))

    cnt_part = _hist(iv_off)
    s_v = _bstats(cnt_part, H1T)
    LT = _lview(H1T, s_v, row(vg), row(vbe), vw2.T, row(vb2))
    LVT = jnp.concatenate([LT.reshape(B * N, DIM), valT2], axis=1)
    aggT = _sc_attn(LVT, iv_off.reshape(-1))

    W2 = jnp.pad(pw1.T, ((0, 125), (0, 64)))
    pw2Tp = jnp.pad(pw2.T, ((0, 64), (0, 0)))
    pgp = jnp.pad(row(pg), ((0, 0), (0, 64)))
    pbep = jnp.pad(row(pbe), ((0, 0), (0, 64)))
    pb1p = jnp.pad(row(pb1), ((0, 0), (0, 64)))

    p1, sp, GPT, cb = _geo1(posgK.reshape(B, NK, 128), posTp, W2, pb1p,
                            pw2Tp, gw1.T, row(pb2), row(gb1))
    keyg3 = keygK.reshape(B, NK, DIM)
    sg = _geo2(keyg3, keyT, p1, sp, gw1.T, GPT, cb, pgp, pbep)
    fgT = _geo3(keyg3, keyT, p1, sg, sp, valT, gw1.T, GPT, cb, gw2.T,
                row(gb2), pw2Tp, row(pb2), row(gg), row(gbe), pgp, pbep)

    out = _final(cos, aggT.reshape(B, M, DIM), fgT, value, Wview,
                 bview[:, None], Wgeo, bgeo[:, None])
    return out


# speedup vs baseline: 1.0271x; 1.0271x over previous
import functools

import jax
import jax.numpy as jnp
from jax import lax
from jax.experimental import pallas as pl
from jax.experimental.pallas import tpu as pltpu
from jax.experimental.pallas import tpu_sc as plsc

B, N, M, C = 2, 1024, 1024, 128
DIM, K, PHD, HID = 256, 16, 64, 1024
NK = N * K
EPS = 1e-5
F32 = jnp.float32
I32 = jnp.int32

NC, NS = 2, 16
NW = NC * NS
RPW = (B * N) // NW

_SC_MESH = plsc.VectorSubcoreMesh(core_axis_name="c", subcore_axis_name="s")


def _prep_body(pos_ref, qimT_ref, kpcT_ref, kpc_ref, valueT_ref,
               WkT_ref, bk_ref, WvT_ref, bv_ref, vw1T_ref, vb1_ref,
               keyT_ref, valT_ref, H1T_ref, cos_ref, negd2_ref):
    keyT = jnp.dot(kpcT_ref[...], WkT_ref[...],
                   preferred_element_type=F32) + bk_ref[...]
    valT = jnp.dot(valueT_ref[...], WvT_ref[...],
                   preferred_element_type=F32) + bv_ref[...]
    keyT_ref[...] = keyT
    valT_ref[...] = valT
    H1T_ref[...] = jnp.dot(keyT, vw1T_ref[...],
                           preferred_element_type=F32) + vb1_ref[...]
    qimT = qimT_ref[...]
    kpc = kpc_ref[...]
    num = jnp.dot(qimT, kpc, preferred_element_type=F32)
    qn = jnp.sqrt(jnp.sum(qimT * qimT, axis=1, keepdims=True))
    kn = jnp.sqrt(jnp.sum(kpc * kpc, axis=0, keepdims=True))
    cos_ref[...] = num / (qn * kn)
    acc = jnp.zeros((N, N), F32)
    for c in range(3):
        pc = pos_ref[c, :]
        d = pc[:, None] - pc[None, :]
        acc = acc + d * d
    negd2_ref[...] = -acc


def _prep(pos, qimT, kpcT, kpc, valueT, WkT, bk2, WvT, bv2, vw1T, vb12):
    full = lambda s: pl.BlockSpec((1,) + s, lambda b: (b, 0, 0))
    w = lambda s: pl.BlockSpec(s, lambda b: (0,) * len(s))
    specs_in = [full((3, N)), full((M, C)), full((N, C)), full((C, N)),
                full((N, C)), w((C, DIM)), w((1, DIM)), w((C, DIM)),
                w((1, DIM)), w((DIM, HID)), w((1, HID))]
    out_shapes = [jax.ShapeDtypeStruct((B, N, DIM), F32),
                  jax.ShapeDtypeStruct((B, N, DIM), F32),
                  jax.ShapeDtypeStruct((B, N, HID), F32),
                  jax.ShapeDtypeStruct((B, M, N), F32),
                  jax.ShapeDtypeStruct((B, N, N), F32)]
    specs_out = [full((N, DIM)), full((N, DIM)), full((N, HID)),
                 full((M, N)), full((N, N))]

    def body(pos_r, qimT_r, kpcT_r, kpc_r, valueT_r, WkT_r, bk_r, WvT_r,
             bv_r, vw1T_r, vb1_r, keyT_r, valT_r, H1T_r, cos_r, negd2_r):
        _prep_body(pos_r.at[0], qimT_r.at[0], kpcT_r.at[0], kpc_r.at[0],
                   valueT_r.at[0], WkT_r, bk_r, WvT_r, bv_r, vw1T_r, vb1_r,
                   keyT_r.at[0], valT_r.at[0], H1T_r.at[0], cos_r.at[0],
                   negd2_r.at[0])

    return pl.pallas_call(
        body, grid=(B,), in_specs=specs_in, out_specs=specs_out,
        out_shape=out_shapes)(pos, qimT, kpcT, kpc, valueT, WkT, bk2, WvT,
                              bv2, vw1T, vb12)


_TOPK_TILE = 256


def _topk_body(x_ref, idx_ref):
    x = x_ref[...]
    it = lax.broadcasted_iota(I32, (_TOPK_TILE, N), 1)
    cols = []
    for _ in range(K):
        m = jnp.max(x, axis=1, keepdims=True)
        cand = jnp.where(x == m, it, N)
        am = jnp.min(cand, axis=1, keepdims=True)
        cols.append(am)
        x = jnp.where(it == am, -jnp.inf, x)
    idx_ref[...] = jnp.concatenate(cols, axis=1)


def _topk(scores):
    rows = scores.shape[0]
    grid = (rows // _TOPK_TILE,)
    return pl.pallas_call(
        _topk_body, grid=grid,
        in_specs=[pl.BlockSpec((_TOPK_TILE, N), lambda i: (i, 0))],
        out_specs=pl.BlockSpec((_TOPK_TILE, K), lambda i: (i, 0)),
        out_shape=jax.ShapeDtypeStruct((rows, K), I32))(scores)


_GCH = 8
_GROWS = 128


def _sc_gather_body(keyT_hbm, posT_hbm, ig_hbm, keyg_hbm, posg_hbm,
                    idx_v, kb0, kb1, pb0, pb1, semg, sems):
    wid = lax.axis_index("s") * NC + lax.axis_index("c")
    base = wid * (_GCH * _GROWS)
    pltpu.sync_copy(ig_hbm.at[pl.ds(base, _GCH * _GROWS)], idx_v)
    kb = [kb0, kb1]
    pb = [pb0, pb1]
    sd = [None] * _GCH
    for c in range(_GCH):
        if c >= 2:
            sd[c - 2][0].wait()
            sd[c - 2][1].wait()
        isl = idx_v.at[pl.ds(c * _GROWS, _GROWS)]
        g1 = pltpu.async_copy(keyT_hbm.at[isl], kb[c % 2], semg)
        g2 = pltpu.async_copy(posT_hbm.at[isl], pb[c % 2], semg)
        g1.wait()
        g2.wait()
        s1 = pltpu.async_copy(kb[c % 2],
                              keyg_hbm.at[pl.ds(base + c * _GROWS, _GROWS)],
                              sems)
        s2 = pltpu.async_copy(pb[c % 2],
                              posg_hbm.at[pl.ds(base + c * _GROWS, _GROWS)],
                              sems)
        sd[c] = (s1, s2)
    for c in (_GCH - 2, _GCH - 1):
        sd[c][0].wait()
        sd[c][1].wait()


@functools.partial(
    pl.kernel, mesh=_SC_MESH,
    out_type=[jax.ShapeDtypeStruct((B * N * K, DIM), F32),
              jax.ShapeDtypeStruct((B * N * K, 128), F32)],
    scratch_types=[pltpu.VMEM((_GCH * _GROWS,), I32),
                   pltpu.VMEM((_GROWS, DIM), F32),
                   pltpu.VMEM((_GROWS, DIM), F32),
                   pltpu.VMEM((_GROWS, 128), F32),
                   pltpu.VMEM((_GROWS, 128), F32),
                   pltpu.SemaphoreType.DMA,
                   pltpu.SemaphoreType.DMA])
def _sc_gather(keyT_hbm, posT_hbm, ig_hbm, keyg_hbm, posg_hbm,
               idx_v, kb0, kb1, pb0, pb1, semg, sems):
    _sc_gather_body(keyT_hbm, posT_hbm, ig_hbm, keyg_hbm, posg_hbm,
                    idx_v, kb0, kb1, pb0, pb1, semg, sems)


_HIST_TILE = 256


def _hist_body(idx_r, cnt_r):
    it = lax.broadcasted_iota(I32, (_HIST_TILE, B * N), 1)
    acc = jnp.zeros((1, B * N), F32)
    idx = idx_r[...]
    for k in range(K):
        eq = (idx[:, k:k + 1] == it).astype(F32)
        acc = acc + jnp.sum(eq, axis=0, keepdims=True)
    blk = jnp.concatenate([acc, jnp.zeros((7, B * N), F32)], axis=0)

    @pl.when(pl.program_id(0) == 0)
    def _():
        cnt_r[...] = blk

    @pl.when(pl.program_id(0) != 0)
    def _():
        cnt_r[...] = cnt_r[...] + blk


def _hist(iv_off):
    grid = ((B * M) // _HIST_TILE,)
    return pl.pallas_call(
        _hist_body, grid=grid,
        in_specs=[pl.BlockSpec((_HIST_TILE, K), lambda i: (i, 0))],
        out_specs=pl.BlockSpec((8, B * N), lambda i: (0, 0)),
        out_shape=jax.ShapeDtypeStruct((8, B * N), F32))(iv_off)


_ACH = 16
_AM = 4


def _attn_rows(lvb, out_v, orow, jm):
    r0 = jm * K
    for g in range(DIM // 16):
        sl = pl.ds(g * 16, 16)
        sv = pl.ds(DIM + g * 16, 16)
        ls = [lvb[r0 + k, sl] for k in range(K)]
        m = ls[0]
        for k in range(1, K):
            m = jnp.maximum(m, ls[k])
        es = [jnp.exp(ls[k] - m) for k in range(K)]
        ssum = es[0]
        for k in range(1, K):
            ssum = ssum + es[k]
        acc = es[0] * lvb[r0, sv]
        for k in range(1, K):
            acc = acc + es[k] * lvb[r0 + k, sv]
        out_v[orow, sl] = acc / ssum


def _sc_attn_body(LVT_hbm, iv_hbm, agg_hbm,
                  idx_v, lvb0, lvb1, out_v, semg):
    wid = lax.axis_index("s") * NC + lax.axis_index("c")
    base = wid * RPW
    rows = _AM * K
    pltpu.sync_copy(iv_hbm.at[pl.ds(base * K, RPW * K)], idx_v)
    pltpu.async_copy(LVT_hbm.at[idx_v.at[pl.ds(0, rows)]], lvb0, semg)

    dummy = LVT_hbm.at[pl.ds(0, rows)]

    def pair(j, carry):
        c0 = 2 * j
        c1 = 2 * j + 1
        pltpu.make_async_copy(dummy, lvb0, semg).wait()
        i1 = idx_v.at[pl.ds(c1 * rows, rows)]
        pltpu.async_copy(LVT_hbm.at[i1], lvb1, semg)

        def m0(jm, cc):
            _attn_rows(lvb0, out_v, c0 * _AM + jm, jm)
            return cc
        lax.fori_loop(0, _AM, m0, 0)

        pltpu.make_async_copy(dummy, lvb1, semg).wait()
        cn = lax.rem(c0 + 2, _ACH)
        i2 = idx_v.at[pl.ds(cn * rows, rows)]
        pltpu.async_copy(LVT_hbm.at[i2], lvb0, semg)

        def m1(jm, cc):
            _attn_rows(lvb1, out_v, c1 * _AM + jm, jm)
            return cc
        lax.fori_loop(0, _AM, m1, 0)
        return carry
    lax.fori_loop(0, _ACH // 2, pair, 0)
    pltpu.make_async_copy(dummy, lvb0, semg).wait()
    pltpu.sync_copy(out_v, agg_hbm.at[pl.ds(base, RPW)])


@functools.partial(
    pl.kernel, mesh=_SC_MESH,
    out_type=jax.ShapeDtypeStruct((B * M, DIM), F32),
    scratch_types=[pltpu.VMEM((RPW * K,), I32),
                   pltpu.VMEM((_AM * K, 2 * DIM), F32),
                   pltpu.VMEM((_AM * K, 2 * DIM), F32),
                   pltpu.VMEM((RPW, DIM), F32),
                   pltpu.SemaphoreType.DMA])
def _sc_attn(LVT_hbm, iv_hbm, agg_hbm, idx_v, lvb0, lvb1, out_v, semg):
    _sc_attn_body(LVT_hbm, iv_hbm, agg_hbm, idx_v, lvb0, lvb1, out_v,
                  semg)


def _bstats_body(cnt_r, H1T_r, s_r):
    cv = cnt_r[0:1, :]
    h = H1T_r.at[0][...]
    s1 = jnp.dot(cv, h, preferred_element_type=F32)
    s2 = jnp.dot(cv, h * h, preferred_element_type=F32)
    blk = jnp.concatenate([s1, s2, jnp.zeros((6, HID), F32)], axis=0)

    @pl.when(pl.program_id(0) == 0)
    def _():
        s_r[...] = blk

    @pl.when(pl.program_id(0) != 0)
    def _():
        s_r[...] = s_r[...] + blk


def _bstats(cnt_part, H1T):
    return pl.pallas_call(
        _bstats_body, grid=(B,),
        in_specs=[pl.BlockSpec((8, N), lambda b: (0, b)),
                  pl.BlockSpec((1, N, HID), lambda b: (b, 0, 0))],
        out_specs=pl.BlockSpec((8, HID), lambda b: (0, 0)),
        out_shape=jax.ShapeDtypeStruct((8, HID), F32))(cnt_part, H1T)


def _lview_body(H1T_r, s_r, vg_r, vbe_r, vw2T_r, vb2_r, LT_r):
    stot = float(B * M * K)
    mu = s_r[0:1, :] / stot
    var = s_r[1:2, :] / stot - mu * mu
    a = vg_r[...] / jnp.sqrt(var + EPS)
    c = vbe_r[...] - mu * a
    H = jnp.maximum(H1T_r.at[0][...] * a + c, 0.0)
    LT_r.at[0][...] = jnp.dot(H, vw2T_r[...],
                              preferred_element_type=F32) + vb2_r[...]


def _lview(H1T, s, vg2, vbe2, vw2T, vb22):
    return pl.pallas_call(
        _lview_body, grid=(B,),
        in_specs=[pl.BlockSpec((1, N, HID), lambda b: (b, 0, 0)),
                  pl.BlockSpec((8, HID), lambda b: (0, 0)),
                  pl.BlockSpec((1, HID), lambda b: (0, 0)),
                  pl.BlockSpec((1, HID), lambda b: (0, 0)),
                  pl.BlockSpec((HID, DIM), lambda b: (0, 0)),
                  pl.BlockSpec((1, DIM), lambda b: (0, 0))],
        out_specs=pl.BlockSpec((1, N, DIM), lambda b: (b, 0, 0)),
        out_shape=jax.ShapeDtypeStruct((B, N, DIM), F32))(
            H1T, s, vg2, vbe2, vw2T, vb22)


def _geo1_body(posg_r, posTp_r, W2_r, pb1_r, pw2Tp_r, gw1T_r, pb2_r,
               gb1_r, p1_r, sp_r, GPT_r, cb_r):
    own = posTp_r.at[0][...]
    pg_ = posg_r.at[0][...]
    posd = (own[:, None, :] - pg_.reshape(N, K, 128)).reshape(NK, 128)
    p1 = jnp.dot(posd, W2_r[...],
                 preferred_element_type=F32) + pb1_r[...]
    p1_r.at[0][...] = p1
    s1 = jnp.sum(p1, axis=0, keepdims=True)
    s2 = jnp.sum(p1 * p1, axis=0, keepdims=True)
    blk = jnp.concatenate([s1, s2, jnp.zeros((6, 128), F32)], axis=0)

    @pl.when(pl.program_id(0) == 0)
    def _():
        sp_r[...] = blk
        GPT_r[...] = jnp.dot(pw2Tp_r[...], gw1T_r[...],
                             preferred_element_type=F32)
        cb_r[...] = jnp.dot(pb2_r[...], gw1T_r[...],
                            preferred_element_type=F32) + gb1_r[...]

    @pl.when(pl.program_id(0) != 0)
    def _():
        sp_r[...] = sp_r[...] + blk


def _geo1(posg3, posTp, W2, pb1p, pw2Tp, gw1T, pb22, gb12):
    return pl.pallas_call(
        _geo1_body, grid=(B,),
        in_specs=[pl.BlockSpec((1, NK, 128), lambda b: (b, 0, 0)),
                  pl.BlockSpec((1, N, 128), lambda b: (b, 0, 0)),
                  pl.BlockSpec((128, 128), lambda b: (0, 0)),
                  pl.BlockSpec((1, 128), lambda b: (0, 0)),
                  pl.BlockSpec((128, DIM), lambda b: (0, 0)),
                  pl.BlockSpec((DIM, HID), lambda b: (0, 0)),
                  pl.BlockSpec((1, DIM), lambda b: (0, 0)),
                  pl.BlockSpec((1, HID), lambda b: (0, 0))],
        out_specs=[pl.BlockSpec((1, NK, 128), lambda b: (b, 0, 0)),
                   pl.BlockSpec((8, 128), lambda b: (0, 0)),
                   pl.BlockSpec((128, HID), lambda b: (0, 0)),
                   pl.BlockSpec((1, HID), lambda b: (0, 0))],
        out_shape=[jax.ShapeDtypeStruct((B, NK, 128), F32),
                   jax.ShapeDtypeStruct((8, 128), F32),
                   jax.ShapeDtypeStruct((128, HID), F32),
                   jax.ShapeDtypeStruct((1, HID), F32)])(
            posg3, posTp, W2, pb1p, pw2Tp, gw1T, pb22, gb12)


_G2T = 1024


def _geo2_body(keyg_r, ownk_r, p1_r, sp_r, gw1T_r, GPT_r, cb_r, pg_r,
               pbe_r, sg_r):
    stot = float(B * NK)
    mup = sp_r[0:1, :] / stot
    varp = sp_r[1:2, :] / stot - mup * mup
    ap = pg_r[...] / jnp.sqrt(varp + EPS)
    cp = pbe_r[...] - mup * ap
    pe = jnp.maximum(p1_r.at[0][...] * ap + cp, 0.0)
    own = ownk_r.at[0][...]
    diff = (own[:, None, :]
            - keyg_r.at[0][...].reshape(_G2T // K, K, DIM)).reshape(
                _G2T, DIM)
    pre1 = (jnp.dot(diff, gw1T_r[...],
                    preferred_element_type=F32)
            + jnp.dot(pe, GPT_r[...], preferred_element_type=F32)
            + cb_r[...])
    s1 = jnp.sum(pre1, axis=0, keepdims=True)
    s2 = jnp.sum(pre1 * pre1, axis=0, keepdims=True)
    blk = jnp.concatenate([s1, s2, jnp.zeros((6, HID), F32)], axis=0)
    first = jnp.logical_and(pl.program_id(0) == 0, pl.program_id(1) == 0)

    @pl.when(first)
    def _():
        sg_r[...] = blk

    @pl.when(jnp.logical_not(first))
    def _():
        sg_r[...] = sg_r[...] + blk


def _geo2(keyg3, keyT, p1, sp, gw1T, GPT, cb, pgp, pbep):
    nt = NK // _G2T
    return pl.pallas_call(
        _geo2_body, grid=(B, nt),
        in_specs=[pl.BlockSpec((1, _G2T, DIM), lambda b, t: (b, t, 0)),
                  pl.BlockSpec((1, _G2T // K, DIM), lambda b, t: (b, t, 0)),
                  pl.BlockSpec((1, _G2T, 128), lambda b, t: (b, t, 0)),
                  pl.BlockSpec((8, 128), lambda b, t: (0, 0)),
                  pl.BlockSpec((DIM, HID), lambda b, t: (0, 0)),
                  pl.BlockSpec((128, HID), lambda b, t: (0, 0)),
                  pl.BlockSpec((1, HID), lambda b, t: (0, 0)),
                  pl.BlockSpec((1, 128), lambda b, t: (0, 0)),
                  pl.BlockSpec((1, 128), lambda b, t: (0, 0))],
        out_specs=pl.BlockSpec((8, HID), lambda b, t: (0, 0)),
        out_shape=jax.ShapeDtypeStruct((8, HID), F32))(
            keyg3, keyT, p1, sp, gw1T, GPT, cb, pgp, pbep)


def _geo3_body(keyg_r, ownk_r, p1_r, sg_r, sp_r, valT_r, gw1T_r, GPT_r,
               cb_r, gw2T_r, gb2_r, pw2Tp_r, pb2_r, gg_r, gbe_r, pg_r,
               pbe_r, fg_r):
    stot = float(B * NK)
    mup = sp_r[0:1, :] / stot
    varp = sp_r[1:2, :] / stot - mup * mup
    ap = pg_r[...] / jnp.sqrt(varp + EPS)
    cp = pbe_r[...] - mup * ap
    pe = jnp.maximum(p1_r.at[0][...] * ap + cp, 0.0)
    own = ownk_r.at[0][...]
    diff = (own[:, None, :]
            - keyg_r.at[0][...].reshape(_G2T // K, K, DIM)).reshape(
                _G2T, DIM)
    pre1 = (jnp.dot(diff, gw1T_r[...], preferred_element_type=F32)
            + jnp.dot(pe, GPT_r[...], preferred_element_type=F32)
            + cb_r[...])
    mug = sg_r[0:1, :] / stot
    varg = sg_r[1:2, :] / stot - mug * mug
    ag = gg_r[...] / jnp.sqrt(varg + EPS)
    cg = gbe_r[...] - mug * ag
    hg = jnp.maximum(pre1 * ag + cg, 0.0)
    logits = jnp.dot(hg, gw2T_r[...],
                     preferred_element_type=F32) + gb2_r[...]
    lg = logits.reshape(_G2T // K, K, DIM)
    mx = jnp.max(lg, axis=1, keepdims=True)
    e = jnp.exp(lg - mx)
    attn = e / jnp.sum(e, axis=1, keepdims=True)
    pos_emb = (jnp.dot(pe, pw2Tp_r[...], preferred_element_type=F32)
               + pb2_r[...]).reshape(_G2T // K, K, DIM)
    v = valT_r.at[0][...]
    fg_r.at[0][...] = (jnp.sum(attn, axis=1) * v
                       + jnp.sum(attn * pos_emb, axis=1))


def _geo3(keyg3, keyT, p1, sg, sp, valT, gw1T, GPT, cb, gw2T, gb22,
          pw2Tp, pb22, gg2, gbe2, pgp, pbep):
    nt = NK // _G2T
    npts = _G2T // K
    return pl.pallas_call(
        _geo3_body, grid=(B, nt),
        in_specs=[pl.BlockSpec((1, _G2T, DIM), lambda b, t: (b, t, 0)),
                  pl.BlockSpec((1, npts, DIM), lambda b, t: (b, t, 0)),
                  pl.BlockSpec((1, _G2T, 128), lambda b, t: (b, t, 0)),
                  pl.BlockSpec((8, HID), lambda b, t: (0, 0)),
                  pl.BlockSpec((8, 128), lambda b, t: (0, 0)),
                  pl.BlockSpec((1, npts, DIM), lambda b, t: (b, t, 0)),
                  pl.BlockSpec((DIM, HID), lambda b, t: (0, 0)),
                  pl.BlockSpec((128, HID), lambda b, t: (0, 0)),
                  pl.BlockSpec((1, HID), lambda b, t: (0, 0)),
                  pl.BlockSpec((HID, DIM), lambda b, t: (0, 0)),
                  pl.BlockSpec((1, DIM), lambda b, t: (0, 0)),
                  pl.BlockSpec((128, DIM), lambda b, t: (0, 0)),
                  pl.BlockSpec((1, DIM), lambda b, t: (0, 0)),
                  pl.BlockSpec((1, HID), lambda b, t: (0, 0)),
                  pl.BlockSpec((1, HID), lambda b, t: (0, 0)),
                  pl.BlockSpec((1, 128), lambda b, t: (0, 0)),
                  pl.BlockSpec((1, 128), lambda b, t: (0, 0))],
        out_specs=pl.BlockSpec((1, npts, DIM), lambda b, t: (b, t, 0)),
        out_shape=jax.ShapeDtypeStruct((B, N, DIM), F32))(
            keyg3, keyT, p1, sg, sp, valT, gw1T, GPT, cb, gw2T, gb22,
            pw2Tp, pb22, gg2, gbe2, pgp, pbep)


def _final_body(cos_r, aggT_r, fgT_r, value_r, Wview_r, bview_r, Wgeo_r,
                bgeo_r, out_r):
    cos = cos_r.at[0][...]
    mx = jnp.max(cos, axis=0, keepdims=True)
    e = jnp.exp(cos - mx)
    s = jnp.sum(e, axis=0, keepdims=True)
    e = e / s
    fvT = lax.dot_general(e, aggT_r.at[0][...],
                          (((0,), (0,)), ((), ())),
                          preferred_element_type=F32)
    pv = lax.dot_general(Wview_r[...], fvT, (((1,), (1,)), ((), ())),
                         preferred_element_type=F32)
    pg_ = lax.dot_general(Wgeo_r[...], fgT_r.at[0][...],
                          (((1,), (1,)), ((), ())),
                          preferred_element_type=F32)
    out_r.at[0][...] = (value_r.at[0][...] + pv + bview_r[...]
                        + pg_ + bgeo_r[...])


def _final(cos, aggT, fgT, value, Wview, bviewC, Wgeo, bgeoC):
    return pl.pallas_call(
        _final_body, grid=(B,),
        in_specs=[pl.BlockSpec((1, M, N), lambda b: (b, 0, 0)),
                  pl.BlockSpec((1, M, DIM), lambda b: (b, 0, 0)),
                  pl.BlockSpec((1, N, DIM), lambda b: (b, 0, 0)),
                  pl.BlockSpec((1, C, N), lambda b: (b, 0, 0)),
                  pl.BlockSpec((C, DIM), lambda b: (0, 0)),
                  pl.BlockSpec((C, 1), lambda b: (0, 0)),
                  pl.BlockSpec((C, DIM), lambda b: (0, 0)),
                  pl.BlockSpec((C, 1), lambda b: (0, 0))],
        out_specs=pl.BlockSpec((1, C, N), lambda b: (b, 0, 0)),
        out_shape=jax.ShapeDtypeStruct((B, C, N), F32))(
            cos, aggT, fgT, value, Wview, bviewC, Wgeo, bgeoC)


def kernel(pos, query_im, key_pc, value, Wk, bk, Wv, bv,
           pw1, pb1, pg, pbe, pw2, pb2,
           vw1, vb1, vg, vbe, vw2, vb2,
           gw1, gb1, gg, gbe, gw2, gb2,
           Wview, bview, Wgeo, bgeo):
    qimT = jnp.transpose(query_im, (0, 2, 1))
    kpcT = jnp.transpose(key_pc, (0, 2, 1))
    valueT = jnp.transpose(value, (0, 2, 1))
    posT = jnp.transpose(pos, (0, 2, 1))
    posTp = jnp.pad(posT, ((0, 0), (0, 0), (0, 125)))

    row = lambda x: x.reshape(1, -1)
    keyT, valT, H1T, cos, negd2 = _prep(
        pos, qimT, kpcT, key_pc, valueT, Wk.T, row(bk), Wv.T, row(bv),
        vw1.T, row(vb1))

    off = ((jnp.arange(B * N, dtype=I32) // N) * N)[:, None]
    iv_off = _topk(cos.reshape(B * M, N)) + off
    ig_off = _topk(negd2.reshape(B * N, N)) + off

    keyT2 = keyT.reshape(B * N, DIM)
    valT2 = valT.reshape(B * N, DIM)
    posT2 = posTp.reshape(B * N, 128)
    keygK, posgK = _sc_gather(keyT2, posT2, ig_off.reshape(-1))

    cnt_part = _hist(iv_off)
    s_v = _bstats(cnt_part, H1T)
    LT = _lview(H1T, s_v, row(vg), row(vbe), vw2.T, row(vb2))
    LVT = jnp.concatenate([LT.reshape(B * N, DIM), valT2], axis=1)
    aggT = _sc_attn(LVT, iv_off.reshape(-1))

    W2 = jnp.pad(pw1.T, ((0, 125), (0, 64)))
    pw2Tp = jnp.pad(pw2.T, ((0, 64), (0, 0)))
    pgp = jnp.pad(row(pg), ((0, 0), (0, 64)))
    pbep = jnp.pad(row(pbe), ((0, 0), (0, 64)))
    pb1p = jnp.pad(row(pb1), ((0, 0), (0, 64)))

    p1, sp, GPT, cb = _geo1(posgK.reshape(B, NK, 128), posTp, W2, pb1p,
                            pw2Tp, gw1.T, row(pb2), row(gb1))
    keyg3 = keygK.reshape(B, NK, DIM)
    sg = _geo2(keyg3, keyT, p1, sp, gw1.T, GPT, cb, pgp, pbep)
    fgT = _geo3(keyg3, keyT, p1, sg, sp, valT, gw1.T, GPT, cb, gw2.T,
                row(gb2), pw2Tp, row(pb2), row(gg), row(gbe), pgp, pbep)

    out = _final(cos, aggT.reshape(B, M, DIM), fgT, value, Wview,
                 bview[:, None], Wgeo, bgeo[:, None])
    return out
